# baseline (speedup 1.0000x reference)
import dataclasses
import functools

import jax
import jax.numpy as jnp
from jax import lax
from jax.experimental import pallas as pl
from jax.experimental.pallas import tpu as pltpu
from jax.experimental.pallas import tpu_sc as plsc

GAMMA = 12.0
NC = 2
NS = 16
NW = NC * NS
LANES = 16
IDX_CHUNK = 128


@functools.lru_cache(maxsize=None)
def _build(B, D):
    assert B % (NW * LANES) == 0 and D % LANES == 0
    bpw = B // NW
    nchunk = bpw // IDX_CHUNK

    mesh = plsc.VectorSubcoreMesh(core_axis_name="c", subcore_axis_name="s")

    cp = pltpu.CompilerParams()
    if "needs_layout_passes" in pltpu.CompilerParams.__dataclass_fields__:
        cp = dataclasses.replace(cp, needs_layout_passes=False)
    if "use_tc_tiling_on_sc" in pltpu.CompilerParams.__dataclass_fields__:
        cp = dataclasses.replace(cp, use_tc_tiling_on_sc=False)

    @functools.partial(
        pl.kernel,
        mesh=mesh,
        compiler_params=cp,
        out_type=jax.ShapeDtypeStruct((B,), jnp.float32),
        scratch_types=[
            pltpu.VMEM((nchunk, IDX_CHUNK), jnp.int32),
            pltpu.VMEM((nchunk, IDX_CHUNK), jnp.int32),
            pltpu.VMEM((nchunk, IDX_CHUNK), jnp.int32),
            pltpu.VMEM((bpw, D), jnp.float32),
            pltpu.VMEM((bpw, D), jnp.float32),
            pltpu.VMEM((bpw, D), jnp.float32),
            pltpu.VMEM((LANES, LANES), jnp.float32),
            pltpu.VMEM((bpw,), jnp.float32),
            pltpu.SemaphoreType.DMA,
        ],
    )
    def kge(ent_hbm, rel_hbm, hidx_hbm, ridx_hbm, tidx_hbm, out_hbm,
            hi_v, ri_v, ti_v, h_v, r_v, t_v, p_v, s_v, sem):
        wid = lax.axis_index("s") * NC + lax.axis_index("c")
        pltpu.sync_copy(hidx_hbm.at[wid], hi_v)
        pltpu.sync_copy(ridx_hbm.at[wid], ri_v)
        pltpu.sync_copy(tidx_hbm.at[wid], ti_v)

        copies = []
        for k in range(nchunk):
            rows = pl.ds(k * IDX_CHUNK, IDX_CHUNK)
            copies.append(pltpu.async_copy(ent_hbm.at[hi_v.at[k]], h_v.at[rows], sem))
            copies.append(pltpu.async_copy(rel_hbm.at[ri_v.at[k]], r_v.at[rows], sem))
            copies.append(pltpu.async_copy(ent_hbm.at[ti_v.at[k]], t_v.at[rows], sem))
        for c in copies:
            c.wait()

        iota16 = lax.iota(jnp.int32, 16)

        @pl.loop(0, bpw, step=LANES)
        def _group(g):
            @pl.loop(0, LANES)
            def _row(i):
                b = g + i
                acc = jnp.abs(h_v[b, pl.ds(0, LANES)]
                              + r_v[b, pl.ds(0, LANES)]
                              - t_v[b, pl.ds(0, LANES)])
                for c in range(1, D // LANES):
                    sl = pl.ds(c * LANES, LANES)
                    acc = acc + jnp.abs(h_v[b, sl] + r_v[b, sl] - t_v[b, sl])
                p_v[i, :] = acc

            tot = plsc.load_gather(p_v, [iota16, jnp.full((16,), 0, jnp.int32)])
            for j in range(1, LANES):
                tot = tot + plsc.load_gather(p_v, [iota16, jnp.full((16,), j, jnp.int32)])
            s_v[pl.ds(g, LANES)] = GAMMA - tot

        pltpu.sync_copy(s_v, out_hbm.at[pl.ds(wid * bpw, bpw)])

    return kge


def kernel(sample, entity_embedding, relation_embedding):
    B = sample.shape[0]
    D = entity_embedding.shape[1]
    ids = sample.astype(jnp.int32)
    bpw = B // NW
    nchunk = bpw // IDX_CHUNK
    hidx = ids[:, 0].reshape(NW, nchunk, IDX_CHUNK)
    ridx = ids[:, 1].reshape(NW, nchunk, IDX_CHUNK)
    tidx = ids[:, 2].reshape(NW, nchunk, IDX_CHUNK)
    score = _build(B, D)(entity_embedding, relation_embedding, hidx, ridx, tidx)
    return score.reshape(B, 1)
